# baseline (speedup 1.0000x reference)
import jax
import jax.numpy as jnp
import numpy as np
from jax import lax
from jax.experimental import pallas as pl
from jax.experimental.pallas import tpu as pltpu
from jax.sharding import Mesh, NamedSharding, PartitionSpec as P

_EPS = 1e-5
_RATES = (6, 12, 18)
_H = 64
_HW = _H * _H
_PAD = 1280
_XPW = _HW + 2 * _PAD


def _taps():
    out = {0: [(0, 0, 0)]}
    t = 1
    for bi, d in enumerate(_RATES, start=1):
        lst = []
        for kh in (-1, 0, 1):
            for kw in (-1, 0, 1):
                lst.append((t, kh * d * _H + kw * d, kw * d))
                t += 1
        out[bi] = lst
    return out


_TAPS = _taps()


def _asppup_kernel(x_ref, wt_ref, bb_ref, wp_ref, bp_ref, o_ref, xs_ref):
    cin = x_ref.shape[1]
    xs_ref[:, :_PAD] = jnp.zeros((cin, _PAD), jnp.bfloat16)
    xs_ref[:, _PAD + _HW:] = jnp.zeros((cin, _PAD), jnp.bfloat16)
    xs_ref[:, _PAD:_PAD + _HW] = x_ref[0].astype(jnp.bfloat16)

    col = lax.broadcasted_iota(jnp.int32, (1, _HW), 1) % _H
    wp = wp_ref[...]
    for br in range(4):
        acc = None
        for (t, s, cw) in _TAPS[br]:
            xs = xs_ref[:, _PAD + s:_PAD + s + _HW]
            y = jnp.dot(wt_ref[t], xs, preferred_element_type=jnp.float32)
            if cw > 0:
                y = jnp.where(col < _H - cw, y, 0.0)
            elif cw < 0:
                y = jnp.where(col >= -cw, y, 0.0)
            acc = y if acc is None else acc + y
        a = jnp.maximum(acc + bb_ref[br, :, 0:1], 0.0)
        z = jnp.dot(wp, a.astype(jnp.bfloat16), preferred_element_type=jnp.float32)
        o_ref[0, br] = jnp.maximum(z + bp_ref[:, 0:1], 0.0)


def kernel(x, w0, w1, w2, w3, wp,
           g0, b0, m0, v0, g1, b1, m1, v1,
           g2, b2, m2, v2, g3, b3, m3, v3,
           gp, bp, mp, vp):
    B, Cin, H, W = x.shape
    Cout = w0.shape[0]

    x2 = x.reshape(B, Cin, H * W)

    def fold(w, g, b, m, v):
        s = g * lax.rsqrt(v + _EPS)
        return w * s[:, None, None, None], b - m * s

    w0f, bias0 = fold(w0, g0, b0, m0, v0)
    rows = [w0f[:, :, 0, 0]]
    biases = [bias0]
    for w, g, b, m, v in ((w1, g1, b1, m1, v1),
                          (w2, g2, b2, m2, v2),
                          (w3, g3, b3, m3, v3)):
        wf, bi = fold(w, g, b, m, v)
        for kh in range(3):
            for kw in range(3):
                rows.append(wf[:, :, kh, kw])
        biases.append(bi)
    wt = jnp.stack(rows).astype(jnp.bfloat16)
    bb = jnp.broadcast_to(jnp.stack(biases)[:, :, None], (4, Cout, 128))
    sp = gp * lax.rsqrt(vp + _EPS)
    wpf = (wp[:, :, 0, 0] * sp[:, None]).astype(jnp.bfloat16)
    bpf = jnp.broadcast_to((bp - mp * sp)[:, None], (Cout, 128))

    def run_block(x2s, wts, bbs, wpfs, bpfs):
        bs = x2s.shape[0]
        return pl.pallas_call(
            _asppup_kernel,
            grid=(bs,),
            in_specs=[
                pl.BlockSpec((1, Cin, _HW), lambda b: (b, 0, 0)),
                pl.BlockSpec((28, Cout, Cin), lambda b: (0, 0, 0)),
                pl.BlockSpec((4, Cout, 128), lambda b: (0, 0, 0)),
                pl.BlockSpec((Cout, Cout), lambda b: (0, 0)),
                pl.BlockSpec((Cout, 128), lambda b: (0, 0)),
            ],
            out_specs=pl.BlockSpec((1, 4, Cout, _HW), lambda b: (b, 0, 0, 0)),
            out_shape=jax.ShapeDtypeStruct((bs, 4, Cout, _HW), jnp.float32),
            scratch_shapes=[pltpu.VMEM((Cin, _XPW), jnp.bfloat16)],
            compiler_params=pltpu.CompilerParams(
                dimension_semantics=("parallel",),
                vmem_limit_bytes=52 * 1024 * 1024,
            ),
        )(x2s, wts, bbs, wpfs, bpfs)

    devs = jax.devices()
    if len(devs) >= 2 and devs[0].platform == "tpu" and B % 2 == 0:
        mesh = Mesh(np.array(devs[:2]), ("d",))
        out = jax.shard_map(
            run_block, mesh=mesh,
            in_specs=(P("d"), P(), P(), P(), P()),
            out_specs=P("d"), check_vma=False)(x2, wt, bb, wpf, bpf)
    else:
        out = run_block(x2, wt, bb, wpf, bpf)

    z = out.reshape(B, 2, 2, Cout, H, W).transpose(0, 3, 4, 1, 5, 2)
    return z.reshape(B, Cout, 2 * H, 2 * W)


# speedup vs baseline: 2.1772x; 2.1772x over previous
import jax
import jax.numpy as jnp
from jax import lax
from jax.experimental import pallas as pl
from jax.experimental.pallas import tpu as pltpu

_EPS = 1e-5
_RATES = (6, 12, 18)
_H = 64
_HW = _H * _H
_PAD = 1280
_XPW = _HW + 2 * _PAD


def _asppup_kernel(x_ref, wt_ref, bb_ref, wp_ref, bp_ref, o_ref, xs_ref):
    cin = x_ref.shape[1]
    xs_ref[:, :_PAD] = jnp.zeros((cin, _PAD), jnp.bfloat16)
    xs_ref[:, _PAD + _HW:] = jnp.zeros((cin, _PAD), jnp.bfloat16)
    xs_ref[:, _PAD:_PAD + _HW] = x_ref[0].astype(jnp.bfloat16)

    col = lax.broadcasted_iota(jnp.int32, (1, _HW), 1) % _H
    wp = wp_ref[...]
    for br in range(4):
        if br == 0:
            acc = jnp.dot(wt_ref[0], xs_ref[:, _PAD:_PAD + _HW],
                          preferred_element_type=jnp.float32)
        else:
            d = _RATES[br - 1]
            base = 1 + 9 * (br - 1)
            acc = None
            for icw, cw in enumerate((-d, 0, d)):
                u = None
                for irh, rh in enumerate((-d, 0, d)):
                    xsl = xs_ref[:, _PAD + rh * _H:_PAD + rh * _H + _HW]
                    y = jnp.dot(wt_ref[base + 3 * irh + icw], xsl,
                                preferred_element_type=jnp.float32)
                    u = y if u is None else u + y
                if cw == 0:
                    g = u
                elif cw > 0:
                    g = jnp.where(col < _H - cw,
                                  jnp.concatenate(
                                      [u[:, cw:], u[:, :cw]], axis=1), 0.0)
                else:
                    g = jnp.where(col >= -cw,
                                  jnp.concatenate(
                                      [u[:, cw:], u[:, :cw]], axis=1), 0.0)
                acc = g if acc is None else acc + g
        a = jnp.maximum(acc + bb_ref[br, :, 0:1], 0.0)
        z = jnp.dot(wp, a.astype(jnp.bfloat16), preferred_element_type=jnp.float32)
        o_ref[0, br] = jnp.maximum(z + bp_ref[:, 0:1], 0.0)


def kernel(x, w0, w1, w2, w3, wp,
           g0, b0, m0, v0, g1, b1, m1, v1,
           g2, b2, m2, v2, g3, b3, m3, v3,
           gp, bp, mp, vp):
    B, Cin, H, W = x.shape
    Cout = w0.shape[0]

    x2 = x.reshape(B, Cin, H * W)

    def fold(w, g, b, m, v):
        s = g * lax.rsqrt(v + _EPS)
        return w * s[:, None, None, None], b - m * s

    w0f, bias0 = fold(w0, g0, b0, m0, v0)
    rows = [w0f[:, :, 0, 0]]
    biases = [bias0]
    for w, g, b, m, v in ((w1, g1, b1, m1, v1),
                          (w2, g2, b2, m2, v2),
                          (w3, g3, b3, m3, v3)):
        wf, bi = fold(w, g, b, m, v)
        for kh in range(3):
            for kw in range(3):
                rows.append(wf[:, :, kh, kw])
        biases.append(bi)
    wt = jnp.stack(rows).astype(jnp.bfloat16)
    bb = jnp.broadcast_to(jnp.stack(biases)[:, :, None], (4, Cout, 128))
    sp = gp * lax.rsqrt(vp + _EPS)
    wpf = (wp[:, :, 0, 0] * sp[:, None]).astype(jnp.bfloat16)
    bpf = jnp.broadcast_to((bp - mp * sp)[:, None], (Cout, 128))

    out = pl.pallas_call(
        _asppup_kernel,
        grid=(B,),
        in_specs=[
            pl.BlockSpec((1, Cin, _HW), lambda b: (b, 0, 0)),
            pl.BlockSpec((28, Cout, Cin), lambda b: (0, 0, 0)),
            pl.BlockSpec((4, Cout, 128), lambda b: (0, 0, 0)),
            pl.BlockSpec((Cout, Cout), lambda b: (0, 0)),
            pl.BlockSpec((Cout, 128), lambda b: (0, 0)),
        ],
        out_specs=pl.BlockSpec((1, 4, Cout, _HW), lambda b: (b, 0, 0, 0)),
        out_shape=jax.ShapeDtypeStruct((B, 4, Cout, _HW), jnp.float32),
        scratch_shapes=[pltpu.VMEM((Cin, _XPW), jnp.bfloat16)],
        compiler_params=pltpu.CompilerParams(
            dimension_semantics=("parallel",),
            vmem_limit_bytes=52 * 1024 * 1024,
        ),
    )(x2, wt, bb, wpf, bpf)

    z = out.reshape(B, 2, 2, Cout, H, W).transpose(0, 3, 4, 1, 5, 2)
    return z.reshape(B, Cout, 2 * H, 2 * W)


# speedup vs baseline: 2.1857x; 1.0039x over previous
import jax
import jax.numpy as jnp
from jax import lax
from jax.experimental import pallas as pl
from jax.experimental.pallas import tpu as pltpu

_EPS = 1e-5
_RATES = (6, 12, 18)
_H = 64
_HW = _H * _H
_PAD = 1280
_XPW = _HW + 2 * _PAD


def _asppup_kernel(x_ref, wt_ref, bb_ref, wp_ref, bp_ref, o_ref, xs_ref):
    cin = x_ref.shape[1]
    xs_ref[:, :_PAD] = jnp.zeros((cin, _PAD), jnp.bfloat16)
    xs_ref[:, _PAD + _HW:] = jnp.zeros((cin, _PAD), jnp.bfloat16)
    xs_ref[:, _PAD:_PAD + _HW] = x_ref[0].astype(jnp.bfloat16)

    col = lax.broadcasted_iota(jnp.int32, (1, _HW), 1) % _H
    wp = wp_ref[...]
    acc0 = None
    row = 0
    for br in range(1, 4):
        d = _RATES[br - 1]
        ys = []
        for irh, rh in enumerate((-d, 0, d)):
            m = 512 if irh == 1 and br == 1 else 384
            xsl = xs_ref[:, _PAD + rh * _H:_PAD + rh * _H + _HW]
            ys.append(jnp.dot(wt_ref[row:row + m, :], xsl,
                              preferred_element_type=jnp.float32))
            if m == 512:
                acc0 = ys[-1][384:512]
            row += m
        acc = None
        for icw, cw in enumerate((-d, 0, d)):
            u = (ys[0][128 * icw:128 * (icw + 1)]
                 + ys[1][128 * icw:128 * (icw + 1)]
                 + ys[2][128 * icw:128 * (icw + 1)])
            if cw == 0:
                g = u
            else:
                rolled = jnp.concatenate([u[:, cw:], u[:, :cw]], axis=1)
                if cw > 0:
                    g = jnp.where(col < _H - cw, rolled, 0.0)
                else:
                    g = jnp.where(col >= -cw, rolled, 0.0)
            acc = g if acc is None else acc + g
        a = jnp.maximum(acc + bb_ref[br, :, 0:1], 0.0)
        z = jnp.dot(wp, a.astype(jnp.bfloat16), preferred_element_type=jnp.float32)
        o_ref[0, br] = jnp.maximum(z + bp_ref[:, 0:1], 0.0)
    a = jnp.maximum(acc0 + bb_ref[0, :, 0:1], 0.0)
    z = jnp.dot(wp, a.astype(jnp.bfloat16), preferred_element_type=jnp.float32)
    o_ref[0, 0] = jnp.maximum(z + bp_ref[:, 0:1], 0.0)


def kernel(x, w0, w1, w2, w3, wp,
           g0, b0, m0, v0, g1, b1, m1, v1,
           g2, b2, m2, v2, g3, b3, m3, v3,
           gp, bp, mp, vp):
    B, Cin, H, W = x.shape
    Cout = w0.shape[0]

    x2 = x.reshape(B, Cin, H * W)

    def fold(w, g, b, m, v):
        s = g * lax.rsqrt(v + _EPS)
        return w * s[:, None, None, None], b - m * s

    w0f, bias0 = fold(w0, g0, b0, m0, v0)
    rows = []
    biases = [bias0]
    for ibr, (w, g, b, m, v) in enumerate(((w1, g1, b1, m1, v1),
                                           (w2, g2, b2, m2, v2),
                                           (w3, g3, b3, m3, v3))):
        wf, bi = fold(w, g, b, m, v)
        for kh in range(3):
            for kw in range(3):
                rows.append(wf[:, :, kh, kw])
            if ibr == 0 and kh == 1:
                rows.append(w0f[:, :, 0, 0])
        biases.append(bi)
    wt = jnp.concatenate(rows, axis=0).astype(jnp.bfloat16)
    bb = jnp.broadcast_to(jnp.stack(biases)[:, :, None], (4, Cout, 128))
    sp = gp * lax.rsqrt(vp + _EPS)
    wpf = (wp[:, :, 0, 0] * sp[:, None]).astype(jnp.bfloat16)
    bpf = jnp.broadcast_to((bp - mp * sp)[:, None], (Cout, 128))

    out = pl.pallas_call(
        _asppup_kernel,
        grid=(B,),
        in_specs=[
            pl.BlockSpec((1, Cin, _HW), lambda b: (b, 0, 0)),
            pl.BlockSpec((28 * Cout, Cin), lambda b: (0, 0)),
            pl.BlockSpec((4, Cout, 128), lambda b: (0, 0, 0)),
            pl.BlockSpec((Cout, Cout), lambda b: (0, 0)),
            pl.BlockSpec((Cout, 128), lambda b: (0, 0)),
        ],
        out_specs=pl.BlockSpec((1, 4, Cout, _HW), lambda b: (b, 0, 0, 0)),
        out_shape=jax.ShapeDtypeStruct((B, 4, Cout, _HW), jnp.float32),
        scratch_shapes=[pltpu.VMEM((Cin, _XPW), jnp.bfloat16)],
        compiler_params=pltpu.CompilerParams(
            dimension_semantics=("parallel",),
            vmem_limit_bytes=52 * 1024 * 1024,
        ),
    )(x2, wt, bb, wpf, bpf)

    z = out.reshape(B, 2, 2, Cout, H, W).transpose(0, 3, 4, 1, 5, 2)
    return z.reshape(B, Cout, 2 * H, 2 * W)


# speedup vs baseline: 2.2569x; 1.0325x over previous
import jax
import jax.numpy as jnp
from jax import lax
from jax.experimental import pallas as pl
from jax.experimental.pallas import tpu as pltpu

_EPS = 1e-5
_RATES = (6, 12, 18)
_H = 64
_HW = _H * _H
_PAD = 1280
_XPW = _HW + 2 * _PAD


def _asppup_kernel(x_ref, wt_ref, bb_ref, wp_ref, bp_ref, o_ref, xs_ref):
    cin = x_ref.shape[1]
    xs_ref[:, :_PAD] = jnp.zeros((cin, _PAD), jnp.bfloat16)
    xs_ref[:, _PAD + _HW:] = jnp.zeros((cin, _PAD), jnp.bfloat16)
    xs_ref[:, _PAD:_PAD + _HW] = x_ref[0].astype(jnp.bfloat16)

    col = lax.broadcasted_iota(jnp.int32, (1, _HW), 1) % _H
    wp = wp_ref[...]
    acc0 = None
    row = 0
    for br in range(1, 4):
        d = _RATES[br - 1]
        ys = []
        for irh, rh in enumerate((-d, 0, d)):
            m = 512 if irh == 1 and br == 1 else 384
            xsl = xs_ref[:, _PAD + rh * _H:_PAD + rh * _H + _HW]
            ys.append(jnp.dot(wt_ref[row:row + m, :], xsl,
                              preferred_element_type=jnp.float32))
            if m == 512:
                acc0 = ys[-1][384:512]
            row += m
        acc = None
        for icw, cw in enumerate((-d, 0, d)):
            u = (ys[0][128 * icw:128 * (icw + 1)]
                 + ys[1][128 * icw:128 * (icw + 1)]
                 + ys[2][128 * icw:128 * (icw + 1)])
            if cw == 0:
                g = u
            else:
                rolled = jnp.concatenate([u[:, cw:], u[:, :cw]], axis=1)
                if cw > 0:
                    g = jnp.where(col < _H - cw, rolled, 0.0)
                else:
                    g = jnp.where(col >= -cw, rolled, 0.0)
            acc = g if acc is None else acc + g
        a = jnp.maximum(acc + bb_ref[br, :, 0:1], 0.0)
        z = jnp.dot(wp, a.astype(jnp.bfloat16), preferred_element_type=jnp.float32)
        o_ref[0, br] = jnp.maximum(z + bp_ref[:, 0:1], 0.0).astype(jnp.bfloat16)
    a = jnp.maximum(acc0 + bb_ref[0, :, 0:1], 0.0)
    z = jnp.dot(wp, a.astype(jnp.bfloat16), preferred_element_type=jnp.float32)
    o_ref[0, 0] = jnp.maximum(z + bp_ref[:, 0:1], 0.0).astype(jnp.bfloat16)


def kernel(x, w0, w1, w2, w3, wp,
           g0, b0, m0, v0, g1, b1, m1, v1,
           g2, b2, m2, v2, g3, b3, m3, v3,
           gp, bp, mp, vp):
    B, Cin, H, W = x.shape
    Cout = w0.shape[0]

    x2 = x.reshape(B, Cin, H * W)

    def fold(w, g, b, m, v):
        s = g * lax.rsqrt(v + _EPS)
        return w * s[:, None, None, None], b - m * s

    w0f, bias0 = fold(w0, g0, b0, m0, v0)
    rows = []
    biases = [bias0]
    for ibr, (w, g, b, m, v) in enumerate(((w1, g1, b1, m1, v1),
                                           (w2, g2, b2, m2, v2),
                                           (w3, g3, b3, m3, v3))):
        wf, bi = fold(w, g, b, m, v)
        for kh in range(3):
            for kw in range(3):
                rows.append(wf[:, :, kh, kw])
            if ibr == 0 and kh == 1:
                rows.append(w0f[:, :, 0, 0])
        biases.append(bi)
    wt = jnp.concatenate(rows, axis=0).astype(jnp.bfloat16)
    bb = jnp.broadcast_to(jnp.stack(biases)[:, :, None], (4, Cout, 128))
    sp = gp * lax.rsqrt(vp + _EPS)
    wpf = (wp[:, :, 0, 0] * sp[:, None]).astype(jnp.bfloat16)
    bpf = jnp.broadcast_to((bp - mp * sp)[:, None], (Cout, 128))

    out = pl.pallas_call(
        _asppup_kernel,
        grid=(B,),
        in_specs=[
            pl.BlockSpec((1, Cin, _HW), lambda b: (b, 0, 0)),
            pl.BlockSpec((28 * Cout, Cin), lambda b: (0, 0)),
            pl.BlockSpec((4, Cout, 128), lambda b: (0, 0, 0)),
            pl.BlockSpec((Cout, Cout), lambda b: (0, 0)),
            pl.BlockSpec((Cout, 128), lambda b: (0, 0)),
        ],
        out_specs=pl.BlockSpec((1, 4, Cout, _HW), lambda b: (b, 0, 0, 0)),
        out_shape=jax.ShapeDtypeStruct((B, 4, Cout, _HW), jnp.bfloat16),
        scratch_shapes=[pltpu.VMEM((Cin, _XPW), jnp.bfloat16)],
        compiler_params=pltpu.CompilerParams(
            dimension_semantics=("parallel",),
            vmem_limit_bytes=52 * 1024 * 1024,
        ),
    )(x2, wt, bb, wpf, bpf)

    z = out.reshape(B, 2, 2, Cout, H, W).transpose(0, 3, 4, 1, 5, 2)
    return z.reshape(B, Cout, 2 * H, 2 * W).astype(jnp.float32)


# speedup vs baseline: 2.3085x; 1.0229x over previous
import jax
import jax.numpy as jnp
from jax import lax
from jax.experimental import pallas as pl
from jax.experimental.pallas import tpu as pltpu

_EPS = 1e-5
_RATES = (6, 12, 18)
_H = 64
_HW = _H * _H
_PAD = 1280
_XPW = _HW + 2 * _PAD


def _asppup_kernel(x_ref, wtb_ref, w0_ref, bb_ref, wp_ref, bp_ref, o_ref,
                   xs_ref, stk_ref):
    cin = x_ref.shape[1]
    xs_ref[:, :_PAD] = jnp.zeros((cin, _PAD), jnp.bfloat16)
    xs_ref[:, _PAD + _HW:] = jnp.zeros((cin, _PAD), jnp.bfloat16)
    xs_ref[:, _PAD:_PAD + _HW] = x_ref[0].astype(jnp.bfloat16)

    for bi, d in enumerate(_RATES):
        for irh, rh in enumerate((-d, 0, d)):
            stk_ref[768 * bi + 256 * irh:768 * bi + 256 * (irh + 1), :] = (
                xs_ref[:, _PAD + rh * _H:_PAD + rh * _H + _HW])

    col = lax.broadcasted_iota(jnp.int32, (1, _HW), 1) % _H
    wp = wp_ref[...]

    def tail(br, acc):
        a = jnp.maximum(acc + bb_ref[br, :, 0:1], 0.0)
        z = jnp.dot(wp, a.astype(jnp.bfloat16),
                    preferred_element_type=jnp.float32)
        o_ref[0, br] = jnp.maximum(z + bp_ref[:, 0:1], 0.0).astype(jnp.bfloat16)

    tail(0, jnp.dot(w0_ref[...], stk_ref[256:512, :],
                    preferred_element_type=jnp.float32))
    for bi, d in enumerate(_RATES):
        y = jnp.dot(wtb_ref[384 * bi:384 * (bi + 1), :],
                    stk_ref[768 * bi:768 * (bi + 1), :],
                    preferred_element_type=jnp.float32)
        acc = None
        for icw, cw in enumerate((-d, 0, d)):
            u = y[128 * icw:128 * (icw + 1)]
            if cw == 0:
                g = u
            else:
                rolled = jnp.concatenate([u[:, cw:], u[:, :cw]], axis=1)
                if cw > 0:
                    g = jnp.where(col < _H - cw, rolled, 0.0)
                else:
                    g = jnp.where(col >= -cw, rolled, 0.0)
            acc = g if acc is None else acc + g
        tail(bi + 1, acc)


def kernel(x, w0, w1, w2, w3, wp,
           g0, b0, m0, v0, g1, b1, m1, v1,
           g2, b2, m2, v2, g3, b3, m3, v3,
           gp, bp, mp, vp):
    B, Cin, H, W = x.shape
    Cout = w0.shape[0]

    x2 = x.reshape(B, Cin, H * W)

    def fold(w, g, b, m, v):
        s = g * lax.rsqrt(v + _EPS)
        return w * s[:, None, None, None], b - m * s

    w0f, bias0 = fold(w0, g0, b0, m0, v0)
    blocks = []
    biases = [bias0]
    for w, g, b, m, v in ((w1, g1, b1, m1, v1),
                          (w2, g2, b2, m2, v2),
                          (w3, g3, b3, m3, v3)):
        wf, bi = fold(w, g, b, m, v)
        blocks.append(jnp.concatenate(
            [jnp.concatenate([wf[:, :, kh, kw] for kh in range(3)], axis=1)
             for kw in range(3)], axis=0))
        biases.append(bi)
    wtb = jnp.concatenate(blocks, axis=0).astype(jnp.bfloat16)
    w0b = w0f[:, :, 0, 0].astype(jnp.bfloat16)
    bb = jnp.broadcast_to(jnp.stack(biases)[:, :, None], (4, Cout, 128))
    sp = gp * lax.rsqrt(vp + _EPS)
    wpf = (wp[:, :, 0, 0] * sp[:, None]).astype(jnp.bfloat16)
    bpf = jnp.broadcast_to((bp - mp * sp)[:, None], (Cout, 128))

    out = pl.pallas_call(
        _asppup_kernel,
        grid=(B,),
        in_specs=[
            pl.BlockSpec((1, Cin, _HW), lambda b: (b, 0, 0)),
            pl.BlockSpec((9 * Cout, 3 * Cin), lambda b: (0, 0)),
            pl.BlockSpec((Cout, Cin), lambda b: (0, 0)),
            pl.BlockSpec((4, Cout, 128), lambda b: (0, 0, 0)),
            pl.BlockSpec((Cout, Cout), lambda b: (0, 0)),
            pl.BlockSpec((Cout, 128), lambda b: (0, 0)),
        ],
        out_specs=pl.BlockSpec((1, 4, Cout, _HW), lambda b: (b, 0, 0, 0)),
        out_shape=jax.ShapeDtypeStruct((B, 4, Cout, _HW), jnp.bfloat16),
        scratch_shapes=[pltpu.VMEM((Cin, _XPW), jnp.bfloat16),
                        pltpu.VMEM((9 * Cin, _HW), jnp.bfloat16)],
        compiler_params=pltpu.CompilerParams(
            dimension_semantics=("parallel",),
            vmem_limit_bytes=52 * 1024 * 1024,
        ),
    )(x2, wtb, w0b, bb, wpf, bpf)

    z = out.reshape(B, 2, 2, Cout, H, W).transpose(0, 3, 4, 1, 5, 2)
    return z.reshape(B, Cout, 2 * H, 2 * W).astype(jnp.float32)
